# Initial kernel scaffold; baseline (speedup 1.0000x reference)
import jax
import jax.numpy as jnp
from jax.experimental import pallas as pl


def kernel(node_attr, edge_index, edge_attr, mlp_W, mlp_b, mlp_g, mlp_be, lin_W, lin_b, gcn_W, gcn_b, bn_g, bn_b):
    raise NotImplementedError("write your pallas kernel here")


# speedup vs baseline: 13.0184x; 13.0184x over previous
import functools

import jax
import jax.numpy as jnp
from jax import lax
from jax.experimental import pallas as pl
from jax.experimental.pallas import tpu as pltpu
from jax.experimental.pallas import tpu_sc as plsc

N = 10000
E = 320000
D = 128
L = 3
EPS = 1e-5

NW = 32
NTILES = 16
CH = 128
EN = E + N
KCH = -(-EN // (NW * CH))
PER_TILE = KCH * CH
EPAD = PER_TILE * NW
NPAD = 10240
DUMMY = N
TROWS = NPAD // NTILES


def _mm(x, w):
    return lax.dot_general(
        x, w, dimension_numbers=(((1,), (0,)), ((), ())),
        precision=lax.Precision.HIGHEST, preferred_element_type=jnp.float32)


def _sc_deg_body(src_hbm, out_hbm, idx_v, ones_v, buf_v, acc_sh):
    c = lax.axis_index("c")
    s = lax.axis_index("s")
    wid = s * 2 + c
    for j in range(TROWS // 16):
        buf_v[pl.ds(16 * j, 16)] = jnp.zeros((16,), jnp.float32)
    for j in range(CH // 16):
        ones_v[pl.ds(16 * j, 16)] = jnp.ones((16,), jnp.float32)
    pltpu.sync_copy(buf_v, acc_sh.at[pl.ds(s * TROWS, TROWS)])
    pltpu.sync_copy(src_hbm.at[wid], idx_v)
    plsc.subcore_barrier()

    def step(j, carry):
        pltpu.sync_copy(ones_v, acc_sh.at[idx_v.at[j]], add=True)
        return carry

    lax.fori_loop(0, KCH, step, 0)
    plsc.subcore_barrier()
    pltpu.sync_copy(acc_sh.at[pl.ds(s * TROWS, TROWS)], buf_v)
    pltpu.sync_copy(buf_v, out_hbm.at[pl.ds(c * NPAD + s * TROWS, TROWS)])


_sc_deg = pl.kernel(
    _sc_deg_body,
    mesh=plsc.VectorSubcoreMesh(core_axis_name="c", subcore_axis_name="s"),
    out_type=jax.ShapeDtypeStruct((2 * NPAD,), jnp.float32),
    scratch_types=[
        pltpu.VMEM((KCH, CH), jnp.int32),
        pltpu.VMEM((CH,), jnp.float32),
        pltpu.VMEM((TROWS,), jnp.float32),
        pltpu.VMEM_SHARED((NPAD,), jnp.float32),
    ],
)


def _sc_agg_body(y_hbm, src_hbm, dst_hbm, out_hbm,
                 sidx_v, didx_v, rows_v, zb_v, sem, acc_sh):
    c = lax.axis_index("c")
    s = lax.axis_index("s")
    wid = s * 2 + c
    for r in range(16):
        for q in range(D // 16):
            zb_v[r, pl.ds(16 * q, 16)] = jnp.zeros((16,), jnp.float32)
    for b in range(TROWS // 16):
        pltpu.sync_copy(zb_v, acc_sh.at[pl.ds(s * TROWS + 16 * b, 16)])
    pltpu.sync_copy(src_hbm.at[wid], sidx_v)
    pltpu.sync_copy(dst_hbm.at[wid], didx_v)
    plsc.subcore_barrier()

    def step(j, carry):
        pltpu.async_copy(y_hbm.at[sidx_v.at[j]], rows_v, sem).wait()
        pltpu.sync_copy(rows_v, acc_sh.at[didx_v.at[j]], add=True)
        return carry

    lax.fori_loop(0, KCH, step, 0)
    plsc.subcore_barrier()
    for b in range(TROWS // CH):
        pltpu.sync_copy(acc_sh.at[pl.ds(s * TROWS + CH * b, CH)], rows_v)
        pltpu.sync_copy(rows_v,
                        out_hbm.at[c, pl.ds(s * TROWS + CH * b, CH)])


_sc_agg = pl.kernel(
    _sc_agg_body,
    mesh=plsc.VectorSubcoreMesh(core_axis_name="c", subcore_axis_name="s"),
    out_type=jax.ShapeDtypeStruct((2, NPAD, D), jnp.float32),
    scratch_types=[
        pltpu.VMEM((KCH, CH), jnp.int32),
        pltpu.VMEM((KCH, CH), jnp.int32),
        pltpu.VMEM((CH, D), jnp.float32),
        pltpu.VMEM((16, D), jnp.float32),
        pltpu.SemaphoreType.DMA,
        pltpu.VMEM_SHARED((NPAD, D), jnp.float32),
    ],
)


def _bn_relu(t, g, b):
    m = jnp.mean(t, axis=0, keepdims=True)
    v = jnp.mean((t - m) ** 2, axis=0, keepdims=True)
    return jnp.maximum((t - m) * lax.rsqrt(v + EPS) * g + b, 0.0)


def _dinv_from(deg_ref):
    deg = (deg_ref[0] + deg_ref[1])[:N]
    return lax.rsqrt(deg)


def _pad_y(y):
    return jnp.concatenate(
        [y, jnp.zeros((NPAD - N, D), jnp.float32)], axis=0)


def _tc_pre_body(x_ref, deg_ref, mlpW_ref, mlpb_ref, mlpg_ref, mlpbe_ref,
                 linW_ref, linb_ref, gcnW_ref, h_ref, y_ref):
    x = x_ref[...]
    t = _mm(x, mlpW_ref[...]) + mlpb_ref[...]
    x0 = _bn_relu(t, mlpg_ref[...], mlpbe_ref[...])
    h_ref[...] = _mm(x0, linW_ref[...]) + linb_ref[...]
    dinv = _dinv_from(deg_ref)
    y_ref[...] = _pad_y(dinv * _mm(x0, gcnW_ref[...]))


def _tc_pre(x, deg2, mlp_W, mlp_b, mlp_g, mlp_be, linW, linb, gcnW):
    return pl.pallas_call(
        _tc_pre_body,
        out_shape=(jax.ShapeDtypeStruct((N, D), jnp.float32),
                   jax.ShapeDtypeStruct((NPAD, D), jnp.float32)),
    )(x, deg2, mlp_W, mlp_b.reshape(1, D), mlp_g.reshape(1, D),
      mlp_be.reshape(1, D), linW, linb.reshape(1, D), gcnW)


def _tc_bn_body(h_ref, p_ref, deg_ref, gcnb_ref, bng_ref, bnb_ref, out_ref):
    dinv = _dinv_from(deg_ref)
    agg = p_ref[0, :N] + p_ref[1, :N]
    g = dinv * agg + gcnb_ref[...]
    t = h_ref[...] + g
    out_ref[...] = _bn_relu(t, bng_ref[...], bnb_ref[...])


def _tc_bn(h, p, deg2, gcnb, bng, bnb):
    return pl.pallas_call(
        _tc_bn_body,
        out_shape=jax.ShapeDtypeStruct((N, D), jnp.float32),
    )(h, p, deg2, gcnb.reshape(1, D), bng.reshape(1, D), bnb.reshape(1, D))


def _tc_mm_body(x_ref, deg_ref, linW_ref, linb_ref, gcnW_ref, h_ref, y_ref):
    x = x_ref[...]
    h_ref[...] = _mm(x, linW_ref[...]) + linb_ref[...]
    dinv = _dinv_from(deg_ref)
    y_ref[...] = _pad_y(dinv * _mm(x, gcnW_ref[...]))


def _tc_mm(x, deg2, linW, linb, gcnW):
    return pl.pallas_call(
        _tc_mm_body,
        out_shape=(jax.ShapeDtypeStruct((N, D), jnp.float32),
                   jax.ShapeDtypeStruct((NPAD, D), jnp.float32)),
    )(x, deg2, linW, linb.reshape(1, D), gcnW)


def kernel(node_attr, edge_index, edge_attr, mlp_W, mlp_b, mlp_g, mlp_be,
           lin_W, lin_b, gcn_W, gcn_b, bn_g, bn_b):
    pad = EPAD - EN
    loops = jnp.arange(N, dtype=jnp.int32)
    fill = jnp.full((pad,), DUMMY, jnp.int32)
    srcp = jnp.concatenate([edge_index[0], loops, fill]).reshape(NW, KCH, CH)
    dstp = jnp.concatenate([edge_index[1], loops, fill]).reshape(NW, KCH, CH)
    degp = _sc_deg(srcp)
    deg2 = degp.reshape(2, NPAD, 1)

    h, y = _tc_pre(node_attr, deg2, mlp_W, mlp_b, mlp_g, mlp_be,
                   lin_W[0], lin_b[0], gcn_W[0])
    for i in range(L):
        p = _sc_agg(y, srcp, dstp)
        x = _tc_bn(h, p, deg2, gcn_b[i], bn_g[i], bn_b[i])
        if i < L - 1:
            h, y = _tc_mm(x, deg2, lin_W[i + 1], lin_b[i + 1], gcn_W[i + 1])
    return x
